# Initial kernel scaffold; baseline (speedup 1.0000x reference)
import jax
import jax.numpy as jnp
from jax.experimental import pallas as pl


def kernel(x, W1, b1, W2, b2):
    raise NotImplementedError("write your pallas kernel here")


# speedup vs baseline: 13.2386x; 13.2386x over previous
import functools

import jax
import jax.numpy as jnp
from jax.experimental import pallas as pl
from jax.experimental.pallas import tpu as pltpu
from jax.experimental.pallas import tpu_sc as plsc

B = 8
L = 2048
D = 128
WINDOW = 9
K_SYM = 5
PATCH_HIGH = 3
PATCH_LOW = 12
ENT_THR_ROBUST = 1.51
NCAND = (L + PATCH_HIGH - 1) // PATCH_HIGH
KP = 704


def _ent_body(xp_ref, ent_ref):
    xp = xp_ref[...]
    counts = []
    for s in range(K_SYM):
        ind = (xp == s).astype(jnp.float32)
        c = ind[:, 0:L]
        for w in range(1, WINDOW):
            c = c + ind[:, w:w + L]
        counts.append(c)
    total = counts[0] + counts[1] + counts[2] + counts[3] + counts[4]
    total = jnp.maximum(total, 1e-12)
    ent = jnp.zeros((B, L), jnp.float32)
    for s in range(K_SYM):
        p = counts[s] / total
        ent = ent - p * jnp.log2(p + 1e-12)
    ent_ref[...] = ent


def _entropy(xp):
    return pl.pallas_call(
        _ent_body,
        out_shape=jax.ShapeDtypeStruct((B, L), jnp.float32),
    )(xp)


def _walk_masks(entropy):
    mesh = plsc.VectorSubcoreMesh(core_axis_name="c", subcore_axis_name="s")
    nworkers = 32

    @functools.partial(
        pl.kernel,
        out_type=[
            jax.ShapeDtypeStruct((nworkers, KP), jnp.float32),
            jax.ShapeDtypeStruct((nworkers, KP), jnp.float32),
        ],
        mesh=mesh,
        scratch_types=[
            pltpu.VMEM((L + 32,), jnp.float32),
            pltpu.VMEM((KP,), jnp.float32),
            pltpu.VMEM((KP,), jnp.float32),
        ],
    )
    def walk(ent_hbm, mh_hbm, ml_hbm, ent_v, mh_v, ml_v):
        wid = jax.lax.axis_index("s") * 2 + jax.lax.axis_index("c")
        row = jax.lax.rem(wid, B)
        pltpu.sync_copy(ent_hbm.at[row], ent_v.at[pl.ds(0, L)])
        zero = jnp.zeros((16,), jnp.float32)

        def zbody(t, carry):
            mh_v[pl.ds(t * 16, 16)] = zero
            ml_v[pl.ds(t * 16, 16)] = zero
            return carry

        jax.lax.fori_loop(0, KP // 16, zbody, 0)
        lane_i = jax.lax.iota(jnp.int32, 16)
        lane0_f = (1 - jnp.minimum(lane_i, 1)).astype(jnp.float32)

        def body(t, carry):
            i, k = carry
            active = i < L
            hi = ent_v[pl.ds(i, 16)][0] > ENT_THR_ROBUST
            act_f = jnp.where(active, 1.0, 0.0)
            hi_f = jnp.where(hi, 1.0, 0.0)
            mh_v[pl.ds(k, 16)] = mh_v[pl.ds(k, 16)] + lane0_f * (hi_f * act_f)
            ml_v[pl.ds(k, 16)] = (ml_v[pl.ds(k, 16)]
                                  + lane0_f * ((1.0 - hi_f) * act_f))
            di = jnp.where(hi, PATCH_HIGH, PATCH_LOW)
            dk = jnp.where(hi, 1, PATCH_LOW // PATCH_HIGH)
            i2 = jnp.where(active, i + di, i)
            k2 = jnp.where(active, k + dk, k)
            return (i2, k2)

        jax.lax.fori_loop(0, NCAND, body, (jnp.int32(0), jnp.int32(0)))
        pltpu.sync_copy(mh_v, mh_hbm.at[wid])
        pltpu.sync_copy(ml_v, ml_hbm.at[wid])

    mh, ml = walk(entropy)
    return mh[:B], ml[:B]


def _feat_body(xa_ref, xb_ref, xc_ref, mh_ref, ml_ref, w1_ref, b1_ref,
               w2_ref, b2_ref, out_ref):
    s3f = xa_ref[...] + xb_ref[...] + xc_ref[...]
    s3 = s3f[:, 0:KP]
    s12 = s3 + s3f[:, 1:KP + 1] + s3f[:, 2:KP + 2] + s3f[:, 3:KP + 3]

    kk = jax.lax.broadcasted_iota(jnp.int32, (B, KP), 1)
    rem = (L - PATCH_HIGH * kk).astype(jnp.float32)
    valid = kk < NCAND
    len3 = jnp.where(valid, jnp.minimum(rem, float(PATCH_HIGH)), 1.0)
    len12 = jnp.where(valid, jnp.minimum(rem, float(PATCH_LOW)), 1.0)
    pmh = s3 / len3
    pml = s12 / len12

    w1 = w1_ref[...]
    b1 = b1_ref[...]
    mh = mh_ref[...]
    ml = ml_ref[...]
    h = (jnp.maximum(pmh[:, :, None] * w1 + b1, 0.0) * mh[:, :, None]
         + jnp.maximum(pml[:, :, None] * w1 + b1, 0.0) * ml[:, :, None])
    s_h = jnp.sum(h, axis=1)
    count = jnp.sum(mh + ml, axis=1)
    out = jax.lax.dot_general(
        s_h, w2_ref[...], (((1,), (0,)), ((), ())),
        preferred_element_type=jnp.float32,
    )
    out_ref[...] = out / count[:, None] + b2_ref[...]


def _features(xa, xb, xc, mh, ml, W1, b1, W2, b2):
    return pl.pallas_call(
        _feat_body,
        out_shape=jax.ShapeDtypeStruct((B, D), jnp.float32),
    )(xa, xb, xc, mh, ml, W1, b1.reshape(1, D), W2, b2.reshape(1, D))


def kernel(x, W1, b1, W2, b2):
    xp = jnp.pad(x, ((0, 0), (4, 4)), constant_values=-1)
    entropy = _entropy(xp)
    mh, ml = _walk_masks(entropy)
    xf = jnp.pad(x.astype(jnp.float32), ((0, 0), (0, 3 * (KP + 4) - L)))
    xa = xf[:, 0::3]
    xb = xf[:, 1::3]
    xc = xf[:, 2::3]
    blt = _features(xa, xb, xc, mh, ml, W1, b1, W2, b2)
    return (blt, entropy)
